# baseline (speedup 1.0000x reference)
import jax
import jax.numpy as jnp
import numpy as np
from jax import lax
from jax.experimental import pallas as pl
from jax.experimental.pallas import tpu as pltpu
from jax.experimental.pallas import tpu_sc as plsc

N = 10000
E = 160000
D = 256
DH = 128
DE = 16
EP = 163840
NT = 16
NC = 2
CH = 128
EPT = EP // NT
NCHUNK = EPT // CH
RPT = 632
ACC_ROWS = NT * RPT
DUMMY = N


def _eproj_body(ef_ref, we_ref, be_ref, out_ref):
    z = jnp.dot(ef_ref[...], we_ref[...], preferred_element_type=jnp.float32)
    zb = (z + be_ref[...]).astype(jnp.bfloat16)
    out_ref[0] = zb[:, :DH]
    out_ref[1] = zb[:, DH:]


def _eproj(ef, W_e, b_e):
    BE = 4000
    return pl.pallas_call(
        _eproj_body,
        grid=(E // BE,),
        in_specs=[
            pl.BlockSpec((BE, DE), lambda i: (i, 0)),
            pl.BlockSpec((DE, D), lambda i: (0, 0)),
            pl.BlockSpec((1, D), lambda i: (0, 0)),
        ],
        out_specs=pl.BlockSpec((NC, BE, DH), lambda i: (0, i, 0)),
        out_shape=jax.ShapeDtypeStruct((NC, EP, DH), jnp.bfloat16),
    )(ef, W_e, b_e.reshape(1, D))


def _sc_agg(node_cat, ep_flat, src_adj, dst_p):
    mesh = plsc.VectorSubcoreMesh(core_axis_name="c", subcore_axis_name="s")

    def body(node_hbm, ep_hbm, srcadj_hbm, dst_hbm, out_hbm,
             sidx, didx, e0, e1, n0, n1, acc,
             se0, se1, sg0, sg1, ss0, ss1,
             si0, si1, si2, sd0, sd1, sd2):
        ebufs = (e0, e1)
        nbufs = (n0, n1)
        se = (se0, se1)
        sg = (sg0, sg1)
        ss = (ss0, ss1)
        si = (si0, si1, si2)
        sd = (sd0, sd1, sd2)
        c = lax.axis_index("c")
        s = lax.axis_index("s")

        def zero_row(i, _):
            for v in range(DH // 16):
                n0[i, pl.ds(v * 16, 16)] = jnp.zeros((16,), jnp.float32)
            return 0
        lax.fori_loop(0, CH, zero_row, 0)
        for k in range(RPT // CH):
            pltpu.sync_copy(n0, acc.at[pl.ds(s * RPT + k * CH, CH)])
        rem = RPT - (RPT // CH) * CH
        if rem:
            pltpu.sync_copy(n0.at[pl.ds(0, rem)],
                            acc.at[pl.ds(s * RPT + (RPT // CH) * CH, rem)])
        plsc.subcore_barrier()

        def ep_src(j):
            off = pl.multiple_of((c * EP + s * EPT + j * CH) // 2, 8)
            return ep_hbm.at[pl.ds(off, CH // 2)]

        def sidx_src(j):
            return srcadj_hbm.at[pl.ds(c * EP + s * EPT + j * CH, CH)]

        def didx_src(j):
            return dst_hbm.at[pl.ds(s * EPT + j * CH, CH)]

        def group(gg, _):
            for u in range(6):
                i = gg * 6 + u
                ia, ib, ic = i, i - 1, i - 2
                qa = u % 3
                qb, db = (u - 1) % 3, (u - 1) % 2
                qc, dc = (u - 2) % 3, (u - 2) % 2

                @pl.when((ia >= 3) & (ia - 3 < NCHUNK))
                def _():
                    pltpu.make_async_copy(
                        nbufs[(u - 3) % 2], acc.at[didx.at[qa]],
                        ss[(u - 3) % 2]).wait()

                @pl.when(ia < NCHUNK)
                def _():
                    pltpu.async_copy(sidx_src(ia), sidx.at[qa], si[qa])
                    pltpu.async_copy(didx_src(ia), didx.at[qa], sd[qa])

                @pl.when((ib >= 0) & (ib < NCHUNK))
                def _():
                    pltpu.make_async_copy(
                        sidx_src(ib), sidx.at[qb], si[qb]).wait()
                    pltpu.async_copy(ep_src(ib), ebufs[db], se[db])
                    pltpu.async_copy(node_hbm.at[sidx.at[qb]], nbufs[db],
                                     sg[db])

                @pl.when((ic >= 0) & (ic < NCHUNK))
                def _():
                    pltpu.make_async_copy(ep_src(ic), ebufs[dc],
                                          se[dc]).wait()
                    pltpu.make_async_copy(node_hbm.at[sidx.at[qc]],
                                          nbufs[dc], sg[dc]).wait()
                    pltpu.make_async_copy(didx_src(ic), didx.at[qc],
                                          sd[qc]).wait()

                    def relu_row(r2, _):
                        for h in range(2):
                            rr = 2 * r2 + h
                            for g in range(DH // 32):
                                ei = ebufs[dc][r2,
                                               pl.ds(h * 64 + 16 * g, 16)]
                                pa = lax.bitcast_convert_type(
                                    ei << 16, jnp.float32)
                                pb = lax.bitcast_convert_type(
                                    ei & jnp.int32(-65536), jnp.float32)
                                sa = pl.ds(32 * g, 16)
                                sb = pl.ds(32 * g + 16, 16)
                                nbufs[dc][rr, sa] = jnp.maximum(
                                    nbufs[dc][rr, sa] + pa, 0.0)
                                nbufs[dc][rr, sb] = jnp.maximum(
                                    nbufs[dc][rr, sb] + pb, 0.0)
                        return 0
                    lax.fori_loop(0, CH // 2, relu_row, 0)
                    pltpu.async_copy(nbufs[dc], acc.at[didx.at[qc]],
                                     ss[dc], add=True)
            return 0
        lax.fori_loop(0, (NCHUNK + 2 + 5) // 6, group, 0)
        plsc.subcore_barrier()

        for k in range(RPT // CH):
            r = s * RPT + k * CH
            pltpu.sync_copy(acc.at[pl.ds(r, CH)],
                            out_hbm.at[pl.ds(c * ACC_ROWS + r, CH)])
        if rem:
            r = s * RPT + (RPT // CH) * CH
            pltpu.sync_copy(acc.at[pl.ds(r, rem)],
                            out_hbm.at[pl.ds(c * ACC_ROWS + r, rem)])

    f = pl.kernel(
        body,
        out_type=jax.ShapeDtypeStruct((NC * ACC_ROWS, DH), jnp.float32),
        mesh=mesh,
        scratch_types=[
            pltpu.VMEM((3, CH), jnp.int32),
            pltpu.VMEM((3, CH), jnp.int32),
            pltpu.VMEM((CH // 2, DH), jnp.int32),
            pltpu.VMEM((CH // 2, DH), jnp.int32),
            pltpu.VMEM((CH, DH), jnp.float32),
            pltpu.VMEM((CH, DH), jnp.float32),
            pltpu.VMEM_SHARED((ACC_ROWS, DH), jnp.float32),
        ] + [pltpu.SemaphoreType.DMA] * 12,
    )
    return f(node_cat, ep_flat, src_adj, dst_p)


def _mlp1_body(eps_ref, alo_ref, ahi_ref, nf_ref, w1_ref, b1_ref,
               z_ref, sum_ref, sq_ref):
    i = pl.program_id(0)
    scale = 1.0 + eps_ref[0]
    rl = alo_ref[...] + scale * nf_ref[:, :DH]
    rh = ahi_ref[...] + scale * nf_ref[:, DH:]
    z = jnp.dot(rl, w1_ref[:DH, :], preferred_element_type=jnp.float32)
    z = z + jnp.dot(rh, w1_ref[DH:, :], preferred_element_type=jnp.float32)
    z = z + b1_ref[...]
    z_ref[...] = z

    @pl.when(i == 0)
    def _():
        sum_ref[...] = jnp.zeros_like(sum_ref)
        sq_ref[...] = jnp.zeros_like(sq_ref)
    sum_ref[...] += jnp.sum(z, axis=0, keepdims=True)
    sq_ref[...] += jnp.sum(z * z, axis=0, keepdims=True)


def _mlp1(eps, agg_lo, agg_hi, node_feats, W1, b1):
    RB = 400
    return pl.pallas_call(
        _mlp1_body,
        grid=(N // RB,),
        in_specs=[
            pl.BlockSpec(memory_space=pltpu.SMEM),
            pl.BlockSpec((RB, DH), lambda i: (i, 0)),
            pl.BlockSpec((RB, DH), lambda i: (i, 0)),
            pl.BlockSpec((RB, D), lambda i: (i, 0)),
            pl.BlockSpec((D, 2 * D), lambda i: (0, 0)),
            pl.BlockSpec((1, 2 * D), lambda i: (0, 0)),
        ],
        out_specs=[
            pl.BlockSpec((RB, 2 * D), lambda i: (i, 0)),
            pl.BlockSpec((1, 2 * D), lambda i: (0, 0)),
            pl.BlockSpec((1, 2 * D), lambda i: (0, 0)),
        ],
        out_shape=[
            jax.ShapeDtypeStruct((N, 2 * D), jnp.float32),
            jax.ShapeDtypeStruct((1, 2 * D), jnp.float32),
            jax.ShapeDtypeStruct((1, 2 * D), jnp.float32),
        ],
    )(eps, agg_lo, agg_hi, node_feats, W1, b1.reshape(1, 2 * D))


def _mlp2_body(z_ref, sum_ref, sq_ref, g_ref, b_ref, w2_ref, b2_ref, out_ref):
    mean = sum_ref[...] * (1.0 / N)
    var = sq_ref[...] * (1.0 / N) - mean * mean
    inv = lax.rsqrt(var + 1e-5)
    sc = g_ref[...] * inv
    sh = b_ref[...] - mean * sc
    a = jnp.maximum(z_ref[...] * sc + sh, 0.0)
    out = jnp.dot(a, w2_ref[...], preferred_element_type=jnp.float32)
    out_ref[...] = out + b2_ref[...]


def _mlp2(z, sums, sqs, gamma, beta, W2, b2):
    RB = 400
    return pl.pallas_call(
        _mlp2_body,
        grid=(N // RB,),
        in_specs=[
            pl.BlockSpec((RB, 2 * D), lambda i: (i, 0)),
            pl.BlockSpec((1, 2 * D), lambda i: (0, 0)),
            pl.BlockSpec((1, 2 * D), lambda i: (0, 0)),
            pl.BlockSpec((1, 2 * D), lambda i: (0, 0)),
            pl.BlockSpec((1, 2 * D), lambda i: (0, 0)),
            pl.BlockSpec((2 * D, D), lambda i: (0, 0)),
            pl.BlockSpec((1, D), lambda i: (0, 0)),
        ],
        out_specs=pl.BlockSpec((RB, D), lambda i: (i, 0)),
        out_shape=jax.ShapeDtypeStruct((N, D), jnp.float32),
    )(z, sums, sqs, gamma.reshape(1, 2 * D), beta.reshape(1, 2 * D),
      W2, b2.reshape(1, D))


def kernel(node_feats, edge_feats, edge_index, W_e, b_e, eps, W1, b1,
           gamma, beta, W2, b2):
    pad = EP - E
    src = edge_index[0]
    dst = edge_index[1]
    src_p = jnp.concatenate([src, jnp.zeros((pad,), jnp.int32)])
    dst_p = jnp.concatenate([dst, jnp.full((pad,), DUMMY, jnp.int32)])
    src_adj = jnp.concatenate([src_p, src_p + N])
    node_cat = jnp.concatenate([node_feats[:, :DH], node_feats[:, DH:]],
                               axis=0)

    perm = np.arange(D).reshape(D // 32, 2, 16).transpose(0, 2, 1).reshape(D)
    ep2 = _eproj(edge_feats, W_e[:, perm], b_e[perm])
    ep_i32 = jax.lax.bitcast_convert_type(
        ep2.reshape(NC * EP, DH // 2, 2), jnp.int32).reshape(
            NC * EP // 2, DH)
    agg2 = _sc_agg(node_cat, ep_i32, src_adj, dst_p)
    agg_lo = agg2[:N]
    agg_hi = agg2[ACC_ROWS:ACC_ROWS + N]
    z, sums, sqs = _mlp1(eps, agg_lo, agg_hi, node_feats, W1, b1)
    return _mlp2(z, sums, sqs, gamma, beta, W2, b2)


# speedup vs baseline: 2.6765x; 2.6765x over previous
import jax
import jax.numpy as jnp
from jax import lax
from jax.experimental import pallas as pl
from jax.experimental.pallas import tpu as pltpu
from jax.experimental.pallas import tpu_sc as plsc

N = 10000
E = 160000
D = 256
DH = 128
DE = 16
EP = 163840
NT = 16
NC = 2
CH = 128
EPT = EP // NT
NCHUNK = EPT // CH
RPT = 632
ACC_ROWS = NT * RPT
DUMMY = N
NBD = 3
NBS = 4
GRP = 12


def _eproj_body(ef_ref, we_ref, be_ref, out_ref):
    z = jnp.dot(ef_ref[...], we_ref[...], preferred_element_type=jnp.float32)
    zb = z + be_ref[...]
    out_ref[0] = zb[:, :DH]
    out_ref[1] = zb[:, DH:]


def _eproj(ef, W_e, b_e):
    BE = 4000
    return pl.pallas_call(
        _eproj_body,
        grid=(E // BE,),
        in_specs=[
            pl.BlockSpec((BE, DE), lambda i: (i, 0)),
            pl.BlockSpec((DE, D), lambda i: (0, 0)),
            pl.BlockSpec((1, D), lambda i: (0, 0)),
        ],
        out_specs=pl.BlockSpec((NC, BE, DH), lambda i: (0, i, 0)),
        out_shape=jax.ShapeDtypeStruct((NC, EP, DH), jnp.float32),
    )(ef, W_e, b_e.reshape(1, D))


def _sc_agg(node_cat, ep_flat, src_adj, dst_p):
    mesh = plsc.VectorSubcoreMesh(core_axis_name="c", subcore_axis_name="s")

    def body(node_hbm, ep_hbm, srcadj_hbm, dst_hbm, out_hbm,
             sidx, didx, n0, n1, n2, acc, *sems):
        nbufs = (n0, n1, n2)
        ssi = sems[0:NBD]
        sep = sems[NBD:2 * NBD]
        sga = sems[2 * NBD:3 * NBD]
        sdi = sems[3 * NBD:3 * NBD + NBS]
        ssc = sems[3 * NBD + NBS:]
        c = lax.axis_index("c")
        s = lax.axis_index("s")

        def zero_row(i, _):
            for v in range(DH // 16):
                n0[i, pl.ds(v * 16, 16)] = jnp.zeros((16,), jnp.float32)
            return 0
        lax.fori_loop(0, CH, zero_row, 0)
        for k in range(RPT // CH):
            pltpu.sync_copy(n0, acc.at[pl.ds(s * RPT + k * CH, CH)])
        rem = RPT - (RPT // CH) * CH
        if rem:
            pltpu.sync_copy(n0.at[pl.ds(0, rem)],
                            acc.at[pl.ds(s * RPT + (RPT // CH) * CH, rem)])
        plsc.subcore_barrier()

        def ep_src(j):
            off = pl.multiple_of(c * EP + s * EPT + j * CH, 8)
            return ep_hbm.at[pl.ds(off, CH)]

        def sidx_src(j):
            return srcadj_hbm.at[pl.ds(c * EP + s * EPT + j * CH, CH)]

        def didx_src(j):
            return dst_hbm.at[pl.ds(s * EPT + j * CH, CH)]

        def group(gg, _):
            for u in range(GRP):
                i = gg * GRP + u
                ia, ib, ic, id_ = i, i - 1, i - 2, i - 3
                nd, dd = (u - 3) % NBD, (u - 3) % NBS

                @pl.when((id_ >= 0) & (id_ < NCHUNK))
                def _():
                    pltpu.make_async_copy(
                        node_hbm.at[sidx.at[nd]], nbufs[nd], sga[nd]).wait()
                    pltpu.make_async_copy(
                        didx_src(id_), didx.at[dd], sdi[dd]).wait()

                    def relu_row(r2, _):
                        for h in range(2):
                            rr = 2 * r2 + h
                            for g in range(DH // 16):
                                sl = pl.ds(16 * g, 16)
                                nbufs[nd][rr, sl] = jnp.maximum(
                                    nbufs[nd][rr, sl], 0.0)
                        return 0
                    lax.fori_loop(0, CH // 2, relu_row, 0)
                    pltpu.async_copy(nbufs[nd], acc.at[didx.at[dd]],
                                     ssc[dd], add=True)

                @pl.when(ia < NCHUNK)
                def _():
                    @pl.when(ia >= NBS)
                    def _():
                        pltpu.make_async_copy(
                            nbufs[(u - 4) % NBD], acc.at[didx.at[u % NBS]],
                            ssc[u % NBS]).wait()
                    pltpu.async_copy(sidx_src(ia), sidx.at[u % NBD],
                                     ssi[u % NBD])
                    pltpu.async_copy(didx_src(ia), didx.at[u % NBS],
                                     sdi[u % NBS])

                @pl.when((ib >= 0) & (ib < NCHUNK))
                def _():
                    qb = (u - 1) % NBD
                    pltpu.async_copy(ep_src(ib), nbufs[qb], sep[qb])

                @pl.when((ic >= 0) & (ic < NCHUNK))
                def _():
                    qc = (u - 2) % NBD
                    pltpu.make_async_copy(
                        ep_src(ic), nbufs[qc], sep[qc]).wait()
                    pltpu.make_async_copy(
                        sidx_src(ic), sidx.at[qc], ssi[qc]).wait()
                    pltpu.async_copy(node_hbm.at[sidx.at[qc]], nbufs[qc],
                                     sga[qc], add=True)
            return 0
        lax.fori_loop(0, (NCHUNK + 3 + GRP - 1) // GRP, group, 0)

        for j in range(NCHUNK - NBS, NCHUNK):
            pltpu.make_async_copy(nbufs[j % NBD], acc.at[didx.at[j % NBS]],
                                  ssc[j % NBS]).wait()
        plsc.subcore_barrier()

        for k in range(RPT // CH):
            r = s * RPT + k * CH
            pltpu.sync_copy(acc.at[pl.ds(r, CH)],
                            out_hbm.at[pl.ds(c * ACC_ROWS + r, CH)])
        if rem:
            r = s * RPT + (RPT // CH) * CH
            pltpu.sync_copy(acc.at[pl.ds(r, rem)],
                            out_hbm.at[pl.ds(c * ACC_ROWS + r, rem)])

    f = pl.kernel(
        body,
        out_type=jax.ShapeDtypeStruct((NC * ACC_ROWS, DH), jnp.float32),
        mesh=mesh,
        scratch_types=[
            pltpu.VMEM((NBD, CH), jnp.int32),
            pltpu.VMEM((NBS, CH), jnp.int32),
            pltpu.VMEM((CH, DH), jnp.float32),
            pltpu.VMEM((CH, DH), jnp.float32),
            pltpu.VMEM((CH, DH), jnp.float32),
            pltpu.VMEM_SHARED((ACC_ROWS, DH), jnp.float32),
        ] + [pltpu.SemaphoreType.DMA] * (3 * NBD + 2 * NBS),
    )
    return f(node_cat, ep_flat, src_adj, dst_p)


def _mlp1_body(eps_ref, alo_ref, ahi_ref, nf_ref, w1_ref, b1_ref,
               z_ref, sum_ref, sq_ref):
    i = pl.program_id(0)
    scale = 1.0 + eps_ref[0]
    rl = alo_ref[...] + scale * nf_ref[:, :DH]
    rh = ahi_ref[...] + scale * nf_ref[:, DH:]
    z = jnp.dot(rl, w1_ref[:DH, :], preferred_element_type=jnp.float32)
    z = z + jnp.dot(rh, w1_ref[DH:, :], preferred_element_type=jnp.float32)
    z = z + b1_ref[...]
    z_ref[...] = z

    @pl.when(i == 0)
    def _():
        sum_ref[...] = jnp.zeros_like(sum_ref)
        sq_ref[...] = jnp.zeros_like(sq_ref)
    sum_ref[...] += jnp.sum(z, axis=0, keepdims=True)
    sq_ref[...] += jnp.sum(z * z, axis=0, keepdims=True)


def _mlp1(eps, agg_lo, agg_hi, node_feats, W1, b1):
    RB = 400
    return pl.pallas_call(
        _mlp1_body,
        grid=(N // RB,),
        in_specs=[
            pl.BlockSpec(memory_space=pltpu.SMEM),
            pl.BlockSpec((RB, DH), lambda i: (i, 0)),
            pl.BlockSpec((RB, DH), lambda i: (i, 0)),
            pl.BlockSpec((RB, D), lambda i: (i, 0)),
            pl.BlockSpec((D, 2 * D), lambda i: (0, 0)),
            pl.BlockSpec((1, 2 * D), lambda i: (0, 0)),
        ],
        out_specs=[
            pl.BlockSpec((RB, 2 * D), lambda i: (i, 0)),
            pl.BlockSpec((1, 2 * D), lambda i: (0, 0)),
            pl.BlockSpec((1, 2 * D), lambda i: (0, 0)),
        ],
        out_shape=[
            jax.ShapeDtypeStruct((N, 2 * D), jnp.float32),
            jax.ShapeDtypeStruct((1, 2 * D), jnp.float32),
            jax.ShapeDtypeStruct((1, 2 * D), jnp.float32),
        ],
    )(eps, agg_lo, agg_hi, node_feats, W1, b1.reshape(1, 2 * D))


def _mlp2_body(z_ref, sum_ref, sq_ref, g_ref, b_ref, w2_ref, b2_ref, out_ref):
    mean = sum_ref[...] * (1.0 / N)
    var = sq_ref[...] * (1.0 / N) - mean * mean
    inv = lax.rsqrt(var + 1e-5)
    sc = g_ref[...] * inv
    sh = b_ref[...] - mean * sc
    a = jnp.maximum(z_ref[...] * sc + sh, 0.0)
    out = jnp.dot(a, w2_ref[...], preferred_element_type=jnp.float32)
    out_ref[...] = out + b2_ref[...]


def _mlp2(z, sums, sqs, gamma, beta, W2, b2):
    RB = 400
    return pl.pallas_call(
        _mlp2_body,
        grid=(N // RB,),
        in_specs=[
            pl.BlockSpec((RB, 2 * D), lambda i: (i, 0)),
            pl.BlockSpec((1, 2 * D), lambda i: (0, 0)),
            pl.BlockSpec((1, 2 * D), lambda i: (0, 0)),
            pl.BlockSpec((1, 2 * D), lambda i: (0, 0)),
            pl.BlockSpec((1, 2 * D), lambda i: (0, 0)),
            pl.BlockSpec((2 * D, D), lambda i: (0, 0)),
            pl.BlockSpec((1, D), lambda i: (0, 0)),
        ],
        out_specs=pl.BlockSpec((RB, D), lambda i: (i, 0)),
        out_shape=jax.ShapeDtypeStruct((N, D), jnp.float32),
    )(z, sums, sqs, gamma.reshape(1, 2 * D), beta.reshape(1, 2 * D),
      W2, b2.reshape(1, D))


def kernel(node_feats, edge_feats, edge_index, W_e, b_e, eps, W1, b1,
           gamma, beta, W2, b2):
    pad = EP - E
    src = edge_index[0]
    dst = edge_index[1]
    src_p = jnp.concatenate([src, jnp.zeros((pad,), jnp.int32)])
    dst_p = jnp.concatenate([dst, jnp.full((pad,), DUMMY, jnp.int32)])
    src_adj = jnp.concatenate([src_p, src_p + N])
    node_cat = jnp.concatenate([node_feats[:, :DH], node_feats[:, DH:]],
                               axis=0)

    ep2 = _eproj(edge_feats, W_e, b_e)
    agg2 = _sc_agg(node_cat, ep2.reshape(NC * EP, DH), src_adj, dst_p)
    agg_lo = agg2[:N]
    agg_hi = agg2[ACC_ROWS:ACC_ROWS + N]
    z, sums, sqs = _mlp1(eps, agg_lo, agg_hi, node_feats, W1, b1)
    return _mlp2(z, sums, sqs, gamma, beta, W2, b2)


# speedup vs baseline: 2.8601x; 1.0686x over previous
import jax
import jax.numpy as jnp
from jax import lax
from jax.experimental import pallas as pl
from jax.experimental.pallas import tpu as pltpu
from jax.experimental.pallas import tpu_sc as plsc

N = 10000
E = 160000
D = 256
DH = 128
DE = 16
EP = 163840
NT = 16
NC = 2
CH = 128
EPT = EP // NT
NCHUNK = EPT // CH
RPT = 632
ACC_ROWS = NT * RPT
DUMMY = N
NBD = 3
NBS = 4
GRP = 12


def _eproj_body(ef_ref, we_ref, be_ref, out_ref):
    z = jnp.dot(ef_ref[...], we_ref[...], preferred_element_type=jnp.float32)
    zb = z + be_ref[...]
    out_ref[0] = zb[:, :DH]
    out_ref[1] = zb[:, DH:]


def _eproj(ef, W_e, b_e):
    BE = 4000
    return pl.pallas_call(
        _eproj_body,
        grid=(E // BE,),
        in_specs=[
            pl.BlockSpec((BE, DE), lambda i: (i, 0)),
            pl.BlockSpec((DE, D), lambda i: (0, 0)),
            pl.BlockSpec((1, D), lambda i: (0, 0)),
        ],
        out_specs=pl.BlockSpec((NC, BE, DH), lambda i: (0, i, 0)),
        out_shape=jax.ShapeDtypeStruct((NC, EP, DH), jnp.float32),
    )(ef, W_e, b_e.reshape(1, D))


def _sc_agg(node_cat, ep_flat, src_adj, dst_p):
    mesh = plsc.VectorSubcoreMesh(core_axis_name="c", subcore_axis_name="s")

    def body(node_hbm, ep_hbm, srcadj_hbm, dst_hbm, out_hbm,
             sidx, didx, n0, n1, n2, acc, *sems):
        nbufs = (n0, n1, n2)
        ssi = sems[0:NBD]
        sep = sems[NBD:2 * NBD]
        sga = sems[2 * NBD:3 * NBD]
        sdi = sems[3 * NBD:3 * NBD + NBS]
        ssc = sems[3 * NBD + NBS:]
        c = lax.axis_index("c")
        s = lax.axis_index("s")

        def zero_row(i, _):
            for v in range(DH // 16):
                n0[i, pl.ds(v * 16, 16)] = jnp.zeros((16,), jnp.float32)
            return 0
        lax.fori_loop(0, CH, zero_row, 0)
        for k in range(RPT // CH):
            pltpu.sync_copy(n0, acc.at[pl.ds(s * RPT + k * CH, CH)])
        rem = RPT - (RPT // CH) * CH
        if rem:
            pltpu.sync_copy(n0.at[pl.ds(0, rem)],
                            acc.at[pl.ds(s * RPT + (RPT // CH) * CH, rem)])
        plsc.subcore_barrier()

        def ep_src(j):
            off = pl.multiple_of(c * EP + s * EPT + j * CH, 8)
            return ep_hbm.at[pl.ds(off, CH)]

        def sidx_src(j):
            return srcadj_hbm.at[pl.ds(c * EP + s * EPT + j * CH, CH)]

        def didx_src(j):
            return dst_hbm.at[pl.ds(s * EPT + j * CH, CH)]

        def group(gg, _):
            for u in range(GRP):
                i = gg * GRP + u
                ia, ib, ic, id_ = i, i - 1, i - 2, i - 3
                nd, dd = (u - 3) % NBD, (u - 3) % NBS

                @pl.when((ic >= 0) & (ic < NCHUNK))
                def _():
                    qc = (u - 2) % NBD
                    pltpu.make_async_copy(
                        ep_src(ic), nbufs[qc], sep[qc]).wait()
                    pltpu.make_async_copy(
                        sidx_src(ic), sidx.at[qc], ssi[qc]).wait()
                    pltpu.async_copy(node_hbm.at[sidx.at[qc]], nbufs[qc],
                                     sga[qc], add=True)

                @pl.when((id_ >= 0) & (id_ < NCHUNK))
                def _():
                    pltpu.make_async_copy(
                        node_hbm.at[sidx.at[nd]], nbufs[nd], sga[nd]).wait()
                    pltpu.make_async_copy(
                        didx_src(id_), didx.at[dd], sdi[dd]).wait()

                    def relu_row(r2, _):
                        for h in range(2):
                            rr = 2 * r2 + h
                            for g in range(DH // 16):
                                sl = pl.ds(16 * g, 16)
                                nbufs[nd][rr, sl] = jnp.maximum(
                                    nbufs[nd][rr, sl], 0.0)
                        return 0
                    lax.fori_loop(0, CH // 2, relu_row, 0)
                    pltpu.async_copy(nbufs[nd], acc.at[didx.at[dd]],
                                     ssc[dd], add=True)

                @pl.when(ia < NCHUNK)
                def _():
                    @pl.when(ia >= NBS)
                    def _():
                        pltpu.make_async_copy(
                            nbufs[(u - 4) % NBD], acc.at[didx.at[u % NBS]],
                            ssc[u % NBS]).wait()
                    pltpu.async_copy(sidx_src(ia), sidx.at[u % NBD],
                                     ssi[u % NBD])
                    pltpu.async_copy(didx_src(ia), didx.at[u % NBS],
                                     sdi[u % NBS])

                @pl.when((ib >= 0) & (ib < NCHUNK))
                def _():
                    qb = (u - 1) % NBD
                    pltpu.async_copy(ep_src(ib), nbufs[qb], sep[qb])
            return 0
        lax.fori_loop(0, (NCHUNK + 3 + GRP - 1) // GRP, group, 0)

        for j in range(NCHUNK - NBS, NCHUNK):
            pltpu.make_async_copy(nbufs[j % NBD], acc.at[didx.at[j % NBS]],
                                  ssc[j % NBS]).wait()
        plsc.subcore_barrier()

        for k in range(RPT // CH):
            r = s * RPT + k * CH
            pltpu.sync_copy(acc.at[pl.ds(r, CH)],
                            out_hbm.at[pl.ds(c * ACC_ROWS + r, CH)])
        if rem:
            r = s * RPT + (RPT // CH) * CH
            pltpu.sync_copy(acc.at[pl.ds(r, rem)],
                            out_hbm.at[pl.ds(c * ACC_ROWS + r, rem)])

    f = pl.kernel(
        body,
        out_type=jax.ShapeDtypeStruct((NC * ACC_ROWS, DH), jnp.float32),
        mesh=mesh,
        scratch_types=[
            pltpu.VMEM((NBD, CH), jnp.int32),
            pltpu.VMEM((NBS, CH), jnp.int32),
            pltpu.VMEM((CH, DH), jnp.float32),
            pltpu.VMEM((CH, DH), jnp.float32),
            pltpu.VMEM((CH, DH), jnp.float32),
            pltpu.VMEM_SHARED((ACC_ROWS, DH), jnp.float32),
        ] + [pltpu.SemaphoreType.DMA] * (3 * NBD + 2 * NBS),
    )
    return f(node_cat, ep_flat, src_adj, dst_p)


def _mlp1_body(eps_ref, alo_ref, ahi_ref, nf_ref, w1_ref, b1_ref,
               z_ref, sum_ref, sq_ref):
    i = pl.program_id(0)
    scale = 1.0 + eps_ref[0]
    rl = alo_ref[...] + scale * nf_ref[:, :DH]
    rh = ahi_ref[...] + scale * nf_ref[:, DH:]
    z = jnp.dot(rl, w1_ref[:DH, :], preferred_element_type=jnp.float32)
    z = z + jnp.dot(rh, w1_ref[DH:, :], preferred_element_type=jnp.float32)
    z = z + b1_ref[...]
    z_ref[...] = z

    @pl.when(i == 0)
    def _():
        sum_ref[...] = jnp.zeros_like(sum_ref)
        sq_ref[...] = jnp.zeros_like(sq_ref)
    sum_ref[...] += jnp.sum(z, axis=0, keepdims=True)
    sq_ref[...] += jnp.sum(z * z, axis=0, keepdims=True)


def _mlp1(eps, agg_lo, agg_hi, node_feats, W1, b1):
    RB = 400
    return pl.pallas_call(
        _mlp1_body,
        grid=(N // RB,),
        in_specs=[
            pl.BlockSpec(memory_space=pltpu.SMEM),
            pl.BlockSpec((RB, DH), lambda i: (i, 0)),
            pl.BlockSpec((RB, DH), lambda i: (i, 0)),
            pl.BlockSpec((RB, D), lambda i: (i, 0)),
            pl.BlockSpec((D, 2 * D), lambda i: (0, 0)),
            pl.BlockSpec((1, 2 * D), lambda i: (0, 0)),
        ],
        out_specs=[
            pl.BlockSpec((RB, 2 * D), lambda i: (i, 0)),
            pl.BlockSpec((1, 2 * D), lambda i: (0, 0)),
            pl.BlockSpec((1, 2 * D), lambda i: (0, 0)),
        ],
        out_shape=[
            jax.ShapeDtypeStruct((N, 2 * D), jnp.float32),
            jax.ShapeDtypeStruct((1, 2 * D), jnp.float32),
            jax.ShapeDtypeStruct((1, 2 * D), jnp.float32),
        ],
    )(eps, agg_lo, agg_hi, node_feats, W1, b1.reshape(1, 2 * D))


def _mlp2_body(z_ref, sum_ref, sq_ref, g_ref, b_ref, w2_ref, b2_ref, out_ref):
    mean = sum_ref[...] * (1.0 / N)
    var = sq_ref[...] * (1.0 / N) - mean * mean
    inv = lax.rsqrt(var + 1e-5)
    sc = g_ref[...] * inv
    sh = b_ref[...] - mean * sc
    a = jnp.maximum(z_ref[...] * sc + sh, 0.0)
    out = jnp.dot(a, w2_ref[...], preferred_element_type=jnp.float32)
    out_ref[...] = out + b2_ref[...]


def _mlp2(z, sums, sqs, gamma, beta, W2, b2):
    RB = 400
    return pl.pallas_call(
        _mlp2_body,
        grid=(N // RB,),
        in_specs=[
            pl.BlockSpec((RB, 2 * D), lambda i: (i, 0)),
            pl.BlockSpec((1, 2 * D), lambda i: (0, 0)),
            pl.BlockSpec((1, 2 * D), lambda i: (0, 0)),
            pl.BlockSpec((1, 2 * D), lambda i: (0, 0)),
            pl.BlockSpec((1, 2 * D), lambda i: (0, 0)),
            pl.BlockSpec((2 * D, D), lambda i: (0, 0)),
            pl.BlockSpec((1, D), lambda i: (0, 0)),
        ],
        out_specs=pl.BlockSpec((RB, D), lambda i: (i, 0)),
        out_shape=jax.ShapeDtypeStruct((N, D), jnp.float32),
    )(z, sums, sqs, gamma.reshape(1, 2 * D), beta.reshape(1, 2 * D),
      W2, b2.reshape(1, D))


def kernel(node_feats, edge_feats, edge_index, W_e, b_e, eps, W1, b1,
           gamma, beta, W2, b2):
    pad = EP - E
    src = edge_index[0]
    dst = edge_index[1]
    src_p = jnp.concatenate([src, jnp.zeros((pad,), jnp.int32)])
    dst_p = jnp.concatenate([dst, jnp.full((pad,), DUMMY, jnp.int32)])
    src_adj = jnp.concatenate([src_p, src_p + N])
    node_cat = jnp.concatenate([node_feats[:, :DH], node_feats[:, DH:]],
                               axis=0)

    ep2 = _eproj(edge_feats, W_e, b_e)
    agg2 = _sc_agg(node_cat, ep2.reshape(NC * EP, DH), src_adj, dst_p)
    agg_lo = agg2[:N]
    agg_hi = agg2[ACC_ROWS:ACC_ROWS + N]
    z, sums, sqs = _mlp1(eps, agg_lo, agg_hi, node_feats, W1, b1)
    return _mlp2(z, sums, sqs, gamma, beta, W2, b2)
